# baseline (speedup 1.0000x reference)
import jax
import jax.numpy as jnp
from jax.experimental import pallas as pl
from jax.experimental.pallas import tpu as pltpu

N = 4096
D = 256
B = 256
NB = N // B


def _mega_kernel(x0_ref, t_ref, s_ref, o_ref, sb_ref, xb_ref, yb_ref, acc_ref):
    q = pl.program_id(0)
    i = pl.program_id(1)
    r = pl.ds(i * B, B)

    @pl.when(q == 0)
    def _init():
        blk = x0_ref[...].astype(jnp.bfloat16)
        acc_ref[r, :] = blk
        xb_ref[r, :] = blk

    @pl.when(q % 2 == 1)
    def _t_phase():
        yb_ref[r, :] = jnp.dot(
            t_ref[...].astype(jnp.bfloat16),
            xb_ref[...],
            preferred_element_type=jnp.float32,
        ).astype(jnp.bfloat16)

    r2 = pl.ds((i // 2) * (2 * B), 2 * B)

    @pl.when(q == 2)
    def _s_cast():
        sb_ref[r, :] = s_ref[...].astype(jnp.bfloat16)

    @pl.when((q == 2) & (i % 2 == 1))
    def _s_stream_dot():
        xn = jnp.dot(sb_ref[r2, :], yb_ref[...], preferred_element_type=jnp.float32)
        xn = xn + xb_ref[r2, :].astype(jnp.float32)
        acc_ref[r2, :] = (acc_ref[r2, :].astype(jnp.float32) + xn).astype(jnp.bfloat16)
        xb_ref[r2, :] = xn.astype(jnp.bfloat16)

    @pl.when((q == 4) & (i % 2 == 0))
    def _s_resident():
        xn = jnp.dot(sb_ref[r2, :], yb_ref[...], preferred_element_type=jnp.float32)
        xn = xn + xb_ref[r2, :].astype(jnp.float32)
        acc_ref[r2, :] = (acc_ref[r2, :].astype(jnp.float32) + xn).astype(jnp.bfloat16)
        xb_ref[r2, :] = xn.astype(jnp.bfloat16)

    @pl.when((q == 6) & (i % 2 == 0))
    def _s_final():
        xn = jnp.dot(sb_ref[r2, :], yb_ref[...], preferred_element_type=jnp.float32)
        xn = xn + xb_ref[r2, :].astype(jnp.float32)
        o_ref[...] = 0.25 * (acc_ref[r2, :].astype(jnp.float32) + xn)


def _x0_idx(q, i):
    return (jnp.where(q == 0, i, NB - 1), 0)


def _t_idx(q, i):
    return (jnp.where(q % 2 == 1, i, jnp.where(q == 0, 0, NB - 1)), 0)


def _s_idx(q, i):
    return (jnp.where(q == 2, i, jnp.where(q < 2, 0, NB - 1)), 0)


def _o_idx(q, i):
    return (jnp.where(q == 6, i // 2, 0), 0)


def kernel(pois_embs, HG_poi_src, HG_poi_tar):
    return pl.pallas_call(
        _mega_kernel,
        grid=(7, NB),
        in_specs=[
            pl.BlockSpec((B, D), _x0_idx),
            pl.BlockSpec((B, N), _t_idx),
            pl.BlockSpec((B, N), _s_idx),
        ],
        out_specs=pl.BlockSpec((2 * B, D), _o_idx),
        out_shape=jax.ShapeDtypeStruct((N, D), jnp.float32),
        scratch_shapes=[
            pltpu.VMEM((N, N), jnp.bfloat16),
            pltpu.VMEM((N, D), jnp.bfloat16),
            pltpu.VMEM((N, D), jnp.bfloat16),
            pltpu.VMEM((N, D), jnp.bfloat16),
        ],
        compiler_params=pltpu.CompilerParams(
            dimension_semantics=("arbitrary", "arbitrary"),
        ),
    )(pois_embs, HG_poi_tar, HG_poi_src)


# speedup vs baseline: 1.0903x; 1.0903x over previous
import jax
import jax.numpy as jnp
from jax.experimental import pallas as pl
from jax.experimental.pallas import tpu as pltpu

N = 4096
D = 256
B = 256
NB = N // B


def _mega_kernel(x0_ref, t_ref, s_ref, o_ref, sb_ref, xb_ref, yb_ref, acc_ref):
    q = pl.program_id(0)
    i = pl.program_id(1)
    r = pl.ds(i * B, B)

    @pl.when(q == 0)
    def _init():
        blk = x0_ref[...].astype(jnp.bfloat16)
        acc_ref[r, :] = blk
        xb_ref[r, :] = blk

    @pl.when((q % 2 == 1) & (i < 8))
    def _t_phase():
        rt = pl.ds(i * 512, 512)
        yb_ref[rt, :] = jnp.dot(
            t_ref[...].astype(jnp.bfloat16),
            xb_ref[...],
            preferred_element_type=jnp.float32,
        ).astype(jnp.bfloat16)

    @pl.when(q == 2)
    def _s_stream():
        st = s_ref[...].astype(jnp.bfloat16)
        sb_ref[r, :] = st
        xn = jnp.dot(st, yb_ref[...], preferred_element_type=jnp.float32)
        xn = xn + xb_ref[r, :].astype(jnp.float32)
        acc_ref[r, :] = (acc_ref[r, :].astype(jnp.float32) + xn).astype(jnp.bfloat16)
        xb_ref[r, :] = xn.astype(jnp.bfloat16)

    r2 = pl.ds((i // 2) * (2 * B), 2 * B)

    @pl.when((q == 4) & (i % 2 == 0))
    def _s_resident():
        xn = jnp.dot(sb_ref[r2, :], yb_ref[...], preferred_element_type=jnp.float32)
        xn = xn + xb_ref[r2, :].astype(jnp.float32)
        acc_ref[r2, :] = (acc_ref[r2, :].astype(jnp.float32) + xn).astype(jnp.bfloat16)
        xb_ref[r2, :] = xn.astype(jnp.bfloat16)

    @pl.when(q == 6)
    def _s_final():
        xn = jnp.dot(sb_ref[r, :], yb_ref[...], preferred_element_type=jnp.float32)
        xn = xn + xb_ref[r, :].astype(jnp.float32)
        o_ref[...] = 0.25 * (acc_ref[r, :].astype(jnp.float32) + xn)


def _x0_idx(q, i):
    return (jnp.where(q == 0, i, NB - 1), 0)


def _t_idx(q, i):
    return (jnp.where(q % 2 == 1, jnp.minimum(i, 7), jnp.where(q == 0, 0, 7)), 0)


def _s_idx(q, i):
    return (jnp.where(q == 2, i, jnp.where(q < 2, 0, NB - 1)), 0)


def _o_idx(q, i):
    return (jnp.where(q == 6, i, 0), 0)


def kernel(pois_embs, HG_poi_src, HG_poi_tar):
    return pl.pallas_call(
        _mega_kernel,
        grid=(7, NB),
        in_specs=[
            pl.BlockSpec((B, D), _x0_idx),
            pl.BlockSpec((2 * B, N), _t_idx),
            pl.BlockSpec((B, N), _s_idx),
        ],
        out_specs=pl.BlockSpec((B, D), _o_idx),
        out_shape=jax.ShapeDtypeStruct((N, D), jnp.float32),
        scratch_shapes=[
            pltpu.VMEM((N, N), jnp.bfloat16),
            pltpu.VMEM((N, D), jnp.bfloat16),
            pltpu.VMEM((N, D), jnp.bfloat16),
            pltpu.VMEM((N, D), jnp.bfloat16),
        ],
        compiler_params=pltpu.CompilerParams(
            dimension_semantics=("arbitrary", "arbitrary"),
            vmem_limit_bytes=67108864,
        ),
    )(pois_embs, HG_poi_tar, HG_poi_src)
